# Initial kernel scaffold; baseline (speedup 1.0000x reference)
import jax
import jax.numpy as jnp
from jax.experimental import pallas as pl


def kernel(pred, partial):
    raise NotImplementedError("write your pallas kernel here")


# speedup vs baseline: 1.6183x; 1.6183x over previous
import jax
import jax.numpy as jnp
from jax.experimental import pallas as pl
from jax.experimental.pallas import tpu as pltpu

_ALPHA = 0.1
_ITERS = 4
_KCHUNK = 128


def _body(pred_ref, part_ref, out_ref, ktb_ref, b2_ref):
    p3 = pred_ref[0]
    kt = part_ref[0]
    n = p3.shape[0]
    m = kt.shape[1]
    nchunks = m // _KCHUNK

    def bf(x):
        return x.astype(jnp.bfloat16).astype(jnp.float32)

    ktb_ref[...] = bf(kt)
    b2_ref[...] = kt[0:1, :] ** 2 + kt[1:2, :] ** 2 + kt[2:3, :] ** 2

    qx = p3[:, 0:1]
    qy = p3[:, 1:2]
    qz = p3[:, 2:3]

    lane = jax.lax.broadcasted_iota(jnp.int32, (n, _KCHUNK), 1)

    for _ in range(_ITERS):
        qxb, qyb, qzb = bf(qx), bf(qy), bf(qz)
        a2 = qx * qx + qy * qy + qz * qz

        def chunk_step(c, carry):
            best, bx, by, bz = carry
            sl = pl.ds(c * _KCHUNK, _KCHUNK)
            kxb = ktb_ref[0:1, sl]
            kyb = ktb_ref[1:2, sl]
            kzb = ktb_ref[2:3, sl]
            kxE = part_ref[0, 0:1, sl]
            kyE = part_ref[0, 1:2, sl]
            kzE = part_ref[0, 2:3, sl]
            dot = qxb * kxb + qyb * kyb + qzb * kzb
            d2 = (a2 + b2_ref[0:1, sl]) - 2.0 * dot
            msk = d2 < best
            best = jnp.where(msk, d2, best)
            bx = jnp.where(msk, jnp.broadcast_to(kxE, (n, _KCHUNK)), bx)
            by = jnp.where(msk, jnp.broadcast_to(kyE, (n, _KCHUNK)), by)
            bz = jnp.where(msk, jnp.broadcast_to(kzE, (n, _KCHUNK)), bz)
            return best, bx, by, bz

        init = (jnp.full((n, _KCHUNK), jnp.inf, jnp.float32),
                jnp.zeros((n, _KCHUNK), jnp.float32),
                jnp.zeros((n, _KCHUNK), jnp.float32),
                jnp.zeros((n, _KCHUNK), jnp.float32))
        best, bx, by, bz = jax.lax.fori_loop(0, nchunks, chunk_step, init)

        bmin = jnp.min(best, axis=1, keepdims=True)
        eq = best == bmin
        li = jnp.min(jnp.where(eq, lane, _KCHUNK), axis=1, keepdims=True)
        pick = lane == li
        cx = jnp.sum(jnp.where(pick, bx, 0.0), axis=1, keepdims=True)
        cy = jnp.sum(jnp.where(pick, by, 0.0), axis=1, keepdims=True)
        cz = jnp.sum(jnp.where(pick, bz, 0.0), axis=1, keepdims=True)

        d = jnp.sqrt(jnp.maximum(bmin, 1e-12))
        dmax = jnp.max(d)
        alpha = _ALPHA * (2.0 - d / (dmax + 1e-6))
        qx = qx + alpha * (cx - qx)
        qy = qy + alpha * (cy - qy)
        qz = qz + alpha * (cz - qz)

    out_ref[0] = jnp.concatenate([qx, qy, qz], axis=1)


@jax.jit
def kernel(pred, partial):
    b, n, _ = pred.shape
    m = partial.shape[1]
    part_t = jnp.swapaxes(partial, 1, 2)
    out = pl.pallas_call(
        _body,
        grid=(b,),
        in_specs=[
            pl.BlockSpec((1, n, 3), lambda i: (i, 0, 0)),
            pl.BlockSpec((1, 3, m), lambda i: (i, 0, 0)),
        ],
        out_specs=pl.BlockSpec((1, n, 3), lambda i: (i, 0, 0)),
        out_shape=jax.ShapeDtypeStruct((b, n, 3), jnp.float32),
        scratch_shapes=[
            pltpu.VMEM((3, m), jnp.float32),
            pltpu.VMEM((1, m), jnp.float32),
        ],
    )(pred, part_t)
    return out
